# baseline (speedup 1.0000x reference)
import jax
import jax.numpy as jnp
from jax.experimental import pallas as pl
from jax.experimental.pallas import tpu as pltpu


def _nconv_block_kernel(x_ref, a_ref, o_ref):
    xb = x_ref[...]
    o = jax.lax.dot_general(
        xb,
        a_ref[...],
        dimension_numbers=(((1,), (0,)), ((), ())),
        preferred_element_type=jnp.float32,
    )
    o_ref[...] = jnp.swapaxes(o, 1, 2).astype(o_ref.dtype)


@jax.jit
def kernel(x, A):
    N, C, V, L = x.shape
    V2, W = A.shape
    assert V == V2
    B = N * C
    x3 = x.reshape(B, V, L)

    tb = min(32, B)
    grid = pl.cdiv(B, tb)
    itemsize = jnp.dtype(x.dtype).itemsize
    lpad = max(L, 128)
    footprint = V * W * itemsize + 2 * tb * (V + W) * lpad * itemsize
    vmem_limit = int(min(48 << 20, max(16 << 20, 2 * footprint)))

    out = pl.pallas_call(
        _nconv_block_kernel,
        out_shape=jax.ShapeDtypeStruct((B, W, L), x.dtype),
        grid=(grid,),
        in_specs=[
            pl.BlockSpec((tb, V, L), lambda i: (i, 0, 0)),
            pl.BlockSpec((V, W), lambda i: (0, 0)),
        ],
        out_specs=pl.BlockSpec((tb, W, L), lambda i: (i, 0, 0)),
        compiler_params=pltpu.CompilerParams(
            dimension_semantics=("parallel",),
            vmem_limit_bytes=vmem_limit,
        ),
    )(x3, A)
    return out.reshape(N, C, W, L)
